# baseline (speedup 1.0000x reference)
import functools

import jax
import jax.numpy as jnp
from jax import lax
from jax.experimental import pallas as pl
from jax.experimental.pallas import tpu as pltpu
from jax.experimental.pallas import tpu_sc as plsc

_B = 2048
_D_IN = 784
_H = 400
_ED = 64
_K = 512


def _enc_kernel(x_ref, w1t_ref, b1_ref, w2t_ref, b2_ref, embt_ref,
                ze_ref, idx_ref):
    xb = x_ref[...]
    h1 = jnp.maximum(
        jnp.dot(xb, w1t_ref[...], preferred_element_type=jnp.float32)
        + b1_ref[...], 0.0)
    z_e = (jnp.dot(h1.astype(jnp.bfloat16), w2t_ref[...],
                   preferred_element_type=jnp.float32) + b2_ref[...])
    embt = embt_ref[...]
    e2 = jnp.sum(embt * embt, axis=0, keepdims=True)
    score = e2 - 2.0 * jnp.dot(z_e, embt, preferred_element_type=jnp.float32)
    min_s = jnp.min(score, axis=1, keepdims=True)
    iota_k = jax.lax.broadcasted_iota(jnp.int32, (_B, _K), 1)
    idx = jnp.min(jnp.where(score == min_s, iota_k, _K), axis=1)
    ze_ref[...] = z_e
    idx_ref[...] = idx[:, None]


def _enc(xb, w1t, b1, w2t, b2, embt):
    return pl.pallas_call(
        _enc_kernel,
        out_shape=[
            jax.ShapeDtypeStruct((_B, _ED), jnp.float32),
            jax.ShapeDtypeStruct((_B, 1), jnp.int32),
        ],
    )(xb, w1t, b1, w2t, b2, embt)


_info = plsc.get_sparse_core_info()
_NC, _NS = _info.num_cores, _info.num_subcores
_NW = _NC * _NS
_BPW = _B // _NW


def _sc_gather_kernel(table_hbm, idx_hbm, out_hbm, idx_v, rows_v, sem):
    wid = lax.axis_index("s") * _NC + lax.axis_index("c")
    base = wid * _BPW
    pltpu.sync_copy(idx_hbm.at[pl.ds(base, _BPW)], idx_v)
    pltpu.async_copy(table_hbm.at[idx_v], rows_v, sem).wait()
    pltpu.sync_copy(rows_v, out_hbm.at[pl.ds(base, _BPW)])


def _sc_gather(emb_padded, idx):
    mesh = plsc.VectorSubcoreMesh(core_axis_name="c", subcore_axis_name="s")
    k = functools.partial(
        pl.kernel, mesh=mesh,
        out_type=jax.ShapeDtypeStruct((_B, 128), jnp.float32),
        scratch_types=[
            pltpu.VMEM((_BPW,), jnp.int32),
            pltpu.VMEM((_BPW, 128), jnp.float32),
            pltpu.SemaphoreType.DMA,
        ],
    )(_sc_gather_kernel)
    return k(emb_padded, idx)


def _dec_kernel(x_ref, ze_ref, zq_ref, w3t_ref, b3_ref, w4t_ref, b4_ref,
                out_ref, bce_ref, vq_ref):
    z_e = ze_ref[...]
    z_q = zq_ref[...][:, :_ED]
    diff = z_e - z_q
    vq_ref[...] = jnp.sum(diff * diff).reshape(1, 1)
    h3 = jnp.maximum(
        jnp.dot(z_q.astype(jnp.bfloat16), w3t_ref[...],
                preferred_element_type=jnp.float32) + b3_ref[...], 0.0)
    logits = (jnp.dot(h3.astype(jnp.bfloat16), w4t_ref[...],
                      preferred_element_type=jnp.float32) + b4_ref[...])
    t = jnp.exp(-logits)
    p = 1.0 / (1.0 + t)
    out_ref[...] = p.astype(jnp.bfloat16)
    x = x_ref[...].astype(jnp.float32)
    bce_ref[...] = jnp.sum((x - 1.0) * logits - jnp.log(1.0 + t)).reshape(1, 1)


def _dec(xb, z_e, z_q, w3t, b3, w4t, b4):
    return pl.pallas_call(
        _dec_kernel,
        out_shape=[
            jax.ShapeDtypeStruct((_B, _D_IN), jnp.bfloat16),
            jax.ShapeDtypeStruct((1, 1), jnp.float32),
            jax.ShapeDtypeStruct((1, 1), jnp.float32),
        ],
    )(xb, z_e, z_q, w3t, b3, w4t, b4)


def kernel(x, W1, b1, W2, b2, W3, b3, W4, b4, emb):
    bf = jnp.bfloat16
    xb = x.astype(bf)
    z_e, idx = _enc(xb, W1.T.astype(bf), b1.reshape(1, -1),
                    W2.T.astype(bf), b2.reshape(1, -1), emb.T)
    emb_padded = jnp.pad(emb, ((0, 0), (0, 128 - _ED)))
    z_q = _sc_gather(emb_padded, idx.reshape(-1))
    out, bce, vq = _dec(xb, z_e, z_q, W3.T.astype(bf), b3.reshape(1, -1),
                        W4.T.astype(bf), b4.reshape(1, -1))
    reconst_loss = -bce[0, 0] / (_B * _D_IN)
    vq_loss = vq[0, 0] / _B
    return out.astype(jnp.float32), reconst_loss, vq_loss, vq_loss
